# baseline (speedup 1.0000x reference)
import jax
import jax.numpy as jnp
from jax.experimental import pallas as pl

_DMODEL = 1024
_NE = 8
_ES = 128
_NT = 2048
_BLK = 256
_NBLK = _NT // _BLK


def _moe_body(x_ref, kmat_ref, vmat_ref, es_ref, out_ref, s_ref, reg_ref):
    i = pl.program_id(0)
    x = x_ref[...]
    sel_raw = jax.lax.dot_general(
        x, es_ref[...], (((1,), (1,)), ((), ())),
        preferred_element_type=jnp.float32)

    m = jnp.max(sel_raw, axis=-1, keepdims=True)
    p = jnp.exp(sel_raw - m)
    p = p / jnp.sum(p, axis=-1, keepdims=True)
    part = jnp.sum(p, axis=0, keepdims=True)

    @pl.when(i == 0)
    def _():
        s_ref[...] = jnp.zeros_like(s_ref)

    s_ref[...] += part

    cols = jax.lax.broadcasted_iota(jnp.int32, sel_raw.shape, 1)
    idx1 = jnp.argmax(sel_raw, axis=-1)[:, None]
    v1 = jnp.max(sel_raw, axis=-1, keepdims=True)
    masked = jnp.where(cols == idx1, -jnp.inf, sel_raw)
    idx2 = jnp.argmax(masked, axis=-1)[:, None]
    v2 = jnp.max(masked, axis=-1, keepdims=True)
    g1 = jax.nn.sigmoid(v1)
    g2 = jax.nn.sigmoid(v2)

    h = jax.lax.dot_general(
        x.astype(jnp.bfloat16), kmat_ref[...], (((1,), (0,)), ((), ())),
        preferred_element_type=jnp.float32)
    h = jnp.maximum(h, 0.0)
    ecol = jax.lax.broadcasted_iota(jnp.int32, h.shape, 1) >> 7
    w = (jnp.where(ecol == idx1, g1, 0.0)
         + jnp.where(ecol == idx2, g2, 0.0))
    h = (h * w).astype(jnp.bfloat16)
    out_ref[...] = jax.lax.dot_general(
        h, vmat_ref[...], (((1,), (0,)), ((), ())),
        preferred_element_type=jnp.float32)

    @pl.when(i == _NBLK - 1)
    def _():
        s = s_ref[...]
        lm = jnp.log(s) - jnp.log(float(_NT))
        reg_ref[...] = jnp.sum(lm * (s / float(_NT)), axis=1, keepdims=True)


def kernel(x, keys, values, expert_sel):
    kmat = keys.transpose(1, 0, 2).reshape(_DMODEL, _NE * _ES).astype(jnp.bfloat16)
    vmat = values.reshape(_NE * _ES, _DMODEL).astype(jnp.bfloat16)
    out, _, reg = pl.pallas_call(
        _moe_body,
        grid=(_NBLK,),
        in_specs=[
            pl.BlockSpec((_BLK, _DMODEL), lambda i: (i, 0)),
            pl.BlockSpec((_DMODEL, _NE * _ES), lambda i: (0, 0)),
            pl.BlockSpec((_NE * _ES, _DMODEL), lambda i: (0, 0)),
            pl.BlockSpec((_NE, _DMODEL), lambda i: (0, 0)),
        ],
        out_specs=[
            pl.BlockSpec((_BLK, _DMODEL), lambda i: (i, 0)),
            pl.BlockSpec((1, _NE), lambda i: (0, 0)),
            pl.BlockSpec((1, 1), lambda i: (0, 0)),
        ],
        out_shape=[
            jax.ShapeDtypeStruct((_NT, _DMODEL), jnp.float32),
            jax.ShapeDtypeStruct((1, _NE), jnp.float32),
            jax.ShapeDtypeStruct((1, 1), jnp.float32),
        ],
    )(x, kmat, vmat, expert_sel)
    return out, reg[0, 0]


# speedup vs baseline: 1.3258x; 1.3258x over previous
import jax
import jax.numpy as jnp
from jax.experimental import pallas as pl
from jax.experimental.pallas import tpu as pltpu

_DMODEL = 1024
_NE = 8
_ES = 128
_NT = 2048
_BLK = 256
_NBLK = _NT // _BLK


def _moe_body(x_ref, keys_ref, values_ref, es_ref, out_ref, s_ref, reg_ref,
              kmat_ref, vmat_ref):
    i = pl.program_id(0)

    @pl.when(i == 0)
    def _():
        s_ref[...] = jnp.zeros_like(s_ref)
        for e in range(_NE):
            kmat_ref[:, e * _ES:(e + 1) * _ES] = (
                keys_ref[e].astype(jnp.bfloat16))
            vmat_ref[e * _ES:(e + 1) * _ES, :] = (
                values_ref[e].astype(jnp.bfloat16))

    x = x_ref[...]
    sel_raw = jax.lax.dot_general(
        x, es_ref[...], (((1,), (1,)), ((), ())),
        preferred_element_type=jnp.float32)

    m = jnp.max(sel_raw, axis=-1, keepdims=True)
    p = jnp.exp(sel_raw - m)
    p = p / jnp.sum(p, axis=-1, keepdims=True)
    s_ref[...] += jnp.sum(p, axis=0, keepdims=True)

    cols = jax.lax.broadcasted_iota(jnp.int32, sel_raw.shape, 1)
    idx1 = jnp.argmax(sel_raw, axis=-1)[:, None]
    v1 = jnp.max(sel_raw, axis=-1, keepdims=True)
    masked = jnp.where(cols == idx1, -jnp.inf, sel_raw)
    idx2 = jnp.argmax(masked, axis=-1)[:, None]
    v2 = jnp.max(masked, axis=-1, keepdims=True)
    g1 = jax.nn.sigmoid(v1)
    g2 = jax.nn.sigmoid(v2)

    h = jax.lax.dot_general(
        x.astype(jnp.bfloat16), kmat_ref[...], (((1,), (0,)), ((), ())),
        preferred_element_type=jnp.float32)
    h = jnp.maximum(h, 0.0)
    ecol = jax.lax.broadcasted_iota(jnp.int32, h.shape, 1) >> 7
    w = (jnp.where(ecol == idx1, g1, 0.0)
         + jnp.where(ecol == idx2, g2, 0.0))
    h = (h * w).astype(jnp.bfloat16)
    out_ref[...] = jax.lax.dot_general(
        h, vmat_ref[...], (((1,), (0,)), ((), ())),
        preferred_element_type=jnp.float32)

    @pl.when(i == _NBLK - 1)
    def _():
        s = s_ref[...]
        lm = jnp.log(s) - jnp.log(float(_NT))
        reg_ref[...] = jnp.sum(lm * (s / float(_NT)), axis=1, keepdims=True)


def kernel(x, keys, values, expert_sel):
    out, _, reg = pl.pallas_call(
        _moe_body,
        grid=(_NBLK,),
        in_specs=[
            pl.BlockSpec((_BLK, _DMODEL), lambda i: (i, 0)),
            pl.BlockSpec((_NE, _DMODEL, _ES), lambda i: (0, 0, 0)),
            pl.BlockSpec((_NE, _ES, _DMODEL), lambda i: (0, 0, 0)),
            pl.BlockSpec((_NE, _DMODEL), lambda i: (0, 0)),
        ],
        out_specs=[
            pl.BlockSpec((_BLK, _DMODEL), lambda i: (i, 0)),
            pl.BlockSpec((1, _NE), lambda i: (0, 0)),
            pl.BlockSpec((1, 1), lambda i: (0, 0)),
        ],
        out_shape=[
            jax.ShapeDtypeStruct((_NT, _DMODEL), jnp.float32),
            jax.ShapeDtypeStruct((1, _NE), jnp.float32),
            jax.ShapeDtypeStruct((1, 1), jnp.float32),
        ],
        scratch_shapes=[
            pltpu.VMEM((_DMODEL, _NE * _ES), jnp.bfloat16),
            pltpu.VMEM((_NE * _ES, _DMODEL), jnp.bfloat16),
        ],
    )(x, keys, values, expert_sel)
    return out, reg[0, 0]


# speedup vs baseline: 1.4016x; 1.0571x over previous
import jax
import jax.numpy as jnp
from jax.experimental import pallas as pl
from jax.experimental.pallas import tpu as pltpu

_DMODEL = 1024
_NE = 8
_ES = 128
_NT = 2048
_BLK = 512
_NBLK = _NT // _BLK


def _moe_body(x_ref, keys_ref, values_ref, es_ref, out_ref, s_ref, reg_ref,
              kmat_ref, vmat_ref):
    i = pl.program_id(0)

    @pl.when(i == 0)
    def _():
        s_ref[...] = jnp.zeros_like(s_ref)
        for e in range(_NE):
            kmat_ref[:, e * _ES:(e + 1) * _ES] = (
                keys_ref[e].astype(jnp.bfloat16))
            vmat_ref[e * _ES:(e + 1) * _ES, :] = (
                values_ref[e].astype(jnp.bfloat16))

    x = x_ref[...]
    sel_raw = jax.lax.dot_general(
        x, es_ref[...], (((1,), (1,)), ((), ())),
        preferred_element_type=jnp.float32)

    m = jnp.max(sel_raw, axis=-1, keepdims=True)
    p = jnp.exp(sel_raw - m)
    p = p / jnp.sum(p, axis=-1, keepdims=True)
    s_ref[...] += jnp.sum(p, axis=0, keepdims=True)

    cols = jax.lax.broadcasted_iota(jnp.int32, sel_raw.shape, 1)
    idx1 = jnp.argmax(sel_raw, axis=-1)[:, None]
    v1 = jnp.max(sel_raw, axis=-1, keepdims=True)
    masked = jnp.where(cols == idx1, -jnp.inf, sel_raw)
    idx2 = jnp.argmax(masked, axis=-1)[:, None]
    v2 = jnp.max(masked, axis=-1, keepdims=True)
    g1 = jax.nn.sigmoid(v1)
    g2 = jax.nn.sigmoid(v2)

    h = jax.lax.dot_general(
        x.astype(jnp.bfloat16), kmat_ref[...], (((1,), (0,)), ((), ())),
        preferred_element_type=jnp.float32)
    h = jnp.maximum(h, 0.0)
    ecol = jax.lax.broadcasted_iota(jnp.int32, h.shape, 1) >> 7
    w = (jnp.where(ecol == idx1, g1, 0.0)
         + jnp.where(ecol == idx2, g2, 0.0))
    h = (h * w).astype(jnp.bfloat16)
    out_ref[...] = jax.lax.dot_general(
        h, vmat_ref[...], (((1,), (0,)), ((), ())),
        preferred_element_type=jnp.float32)

    @pl.when(i == _NBLK - 1)
    def _():
        s = s_ref[...]
        lm = jnp.log(s) - jnp.log(float(_NT))
        reg_ref[...] = jnp.sum(lm * (s / float(_NT)), axis=1, keepdims=True)


def kernel(x, keys, values, expert_sel):
    out, _, reg = pl.pallas_call(
        _moe_body,
        grid=(_NBLK,),
        in_specs=[
            pl.BlockSpec((_BLK, _DMODEL), lambda i: (i, 0)),
            pl.BlockSpec((_NE, _DMODEL, _ES), lambda i: (0, 0, 0)),
            pl.BlockSpec((_NE, _ES, _DMODEL), lambda i: (0, 0, 0)),
            pl.BlockSpec((_NE, _DMODEL), lambda i: (0, 0)),
        ],
        out_specs=[
            pl.BlockSpec((_BLK, _DMODEL), lambda i: (i, 0)),
            pl.BlockSpec((1, _NE), lambda i: (0, 0)),
            pl.BlockSpec((1, 1), lambda i: (0, 0)),
        ],
        out_shape=[
            jax.ShapeDtypeStruct((_NT, _DMODEL), jnp.float32),
            jax.ShapeDtypeStruct((1, _NE), jnp.float32),
            jax.ShapeDtypeStruct((1, 1), jnp.float32),
        ],
        scratch_shapes=[
            pltpu.VMEM((_DMODEL, _NE * _ES), jnp.bfloat16),
            pltpu.VMEM((_NE * _ES, _DMODEL), jnp.bfloat16),
        ],
    )(x, keys, values, expert_sel)
    return out, reg[0, 0]
